# baseline (speedup 1.0000x reference)
import functools

import jax
import jax.numpy as jnp
from jax import lax
from jax.experimental import pallas as pl
from jax.experimental.pallas import tpu as pltpu
from jax.experimental.pallas import tpu_sc as plsc

B = 1024
K = 1000
EMB = 64
C0 = 512
C1 = K - C0


def _sc_kernel_make():
    info = plsc.get_sparse_core_info()
    nc, ns = info.num_cores, info.num_subcores
    nw = nc * ns
    bw = B // nw

    mesh = plsc.VectorSubcoreMesh(core_axis_name="c", subcore_axis_name="s")

    @functools.partial(
        pl.kernel,
        mesh=mesh,
        compiler_params=pltpu.CompilerParams(
            needs_layout_passes=False, use_tc_tiling_on_sc=False),
        out_type=[
            jax.ShapeDtypeStruct((B,), jnp.float32),
            jax.ShapeDtypeStruct((B,), jnp.float32),
        ],
        scratch_types=[
            pltpu.VMEM((bw,), jnp.int32),
            pltpu.VMEM((bw,), jnp.int32),
            pltpu.VMEM((bw, K), jnp.int32),
            pltpu.VMEM((bw, EMB), jnp.float32),
            pltpu.VMEM((bw, EMB), jnp.float32),
            pltpu.VMEM((C0, EMB), jnp.bfloat16),
            pltpu.VMEM((C0, EMB), jnp.bfloat16),
            pltpu.VMEM((C0, EMB), jnp.bfloat16),
            pltpu.VMEM((C0, EMB), jnp.bfloat16),
            pltpu.VMEM((16, 16), jnp.float32),
            pltpu.VMEM((bw, 16), jnp.float32),
            pltpu.VMEM((bw,), jnp.float32),
            pltpu.VMEM((bw,), jnp.float32),
            pltpu.SemaphoreType.DMA,
            pltpu.SemaphoreType.DMA,
            pltpu.SemaphoreType.DMA,
            pltpu.SemaphoreType.DMA,
            pltpu.SemaphoreType.DMA,
        ],
    )
    def sc_kernel(cidx_hbm, tidx_hbm, av_hbm, v_hbm, u_hbm, u16_hbm,
                  scores_hbm, denom_hbm,
                  cidx_v, tidx_v, av_v, crows_v, trows_v,
                  rbuf0, rbuf1, rbuf2, rbuf3, qbuf, accbuf, sc_v, dn_v,
                  sem_s, sem0, sem1, sem2, sem3):
        wid = lax.axis_index("s") * nc + lax.axis_index("c")
        base_b = wid * bw
        lanes = lax.iota(jnp.int32, 16)

        def col(l):
            return jnp.full((16,), l, jnp.int32)

        cp_c = pltpu.make_async_copy(cidx_hbm.at[pl.ds(base_b, bw)], cidx_v, sem_s)
        cp_t = pltpu.make_async_copy(tidx_hbm.at[pl.ds(base_b, bw)], tidx_v, sem_s)
        cp_a = pltpu.make_async_copy(av_hbm.at[pl.ds(base_b, bw)], av_v, sem_s)
        cp_c.start(); cp_t.start(); cp_a.start()
        cp_c.wait(); cp_t.wait(); cp_a.wait()
        cp_cr = pltpu.make_async_copy(v_hbm.at[cidx_v], crows_v, sem_s)
        cp_tr = pltpu.make_async_copy(u_hbm.at[tidx_v], trows_v, sem_s)
        cp_cr.start(); cp_tr.start()

        rbufs = ((rbuf0, rbuf1), (rbuf2, rbuf3))
        sems = ((sem0, sem1), (sem2, sem3))

        def start_gather(lb, t, buf, sem):
            if t == 0:
                src = u16_hbm.at[av_v.at[lb, pl.ds(0, C0)]]
                pltpu.make_async_copy(src, buf, sem).start()
            else:
                src = u16_hbm.at[av_v.at[lb, pl.ds(C0, C1)]]
                pltpu.make_async_copy(src, buf.at[pl.ds(0, C1)], sem).start()

        def wait_gather(t, buf, sem):
            if t == 0:
                src = u16_hbm.at[av_v.at[0, pl.ds(0, C0)]]
                pltpu.make_async_copy(src, buf, sem).wait()
            else:
                src = u16_hbm.at[av_v.at[0, pl.ds(C0, C1)]]
                pltpu.make_async_copy(src, buf.at[pl.ds(0, C1)], sem).wait()

        start_gather(0, 0, rbuf0, sem0)
        start_gather(0, 1, rbuf1, sem1)
        start_gather(1, 0, rbuf2, sem2)
        start_gather(1, 1, rbuf3, sem3)
        cp_cr.wait(); cp_tr.wait()

        hi_mask = jnp.full((16,), 0xFFFF0000, jnp.uint32)

        def unpack_bf16(v32):
            w = plsc.bitcast(v32, jnp.uint32)
            even = plsc.bitcast(w << 16, jnp.float32)
            odd = plsc.bitcast(w & hi_mask, jnp.float32)
            return even, odd

        def compute_chunk(lb, t, rbuf, acc):
            ce0 = plsc.load_gather(crows_v, [col(lb), lanes * 2])
            co0 = plsc.load_gather(crows_v, [col(lb), lanes * 2 + 1])
            ce1 = plsc.load_gather(crows_v, [col(lb), lanes * 2 + 32])
            co1 = plsc.load_gather(crows_v, [col(lb), lanes * 2 + 33])

            def group(gi, acc):
                for r in range(16):
                    row = gi * 16 + r
                    e0, o0 = unpack_bf16(rbuf[row, pl.ds(0, 32)])
                    e1, o1 = unpack_bf16(rbuf[row, pl.ds(32, 32)])
                    q = e0 * ce0
                    q = q + o0 * co0
                    q = q + e1 * ce1
                    q = q + o1 * co1
                    qbuf[r] = q
                d = jnp.zeros((16,), jnp.float32)
                for l in range(16):
                    d = d + plsc.load_gather(qbuf, [lanes, col(l)])
                e = jnp.exp(d)
                if t == 1:
                    e = jnp.where(gi * 16 + lanes < C1, e, jnp.float32(0.0))
                return acc + e

            ngroups = C0 // 16 if t == 0 else (C1 + 15) // 16
            return lax.fori_loop(0, ngroups, group, acc)

        def body(i, acc):
            for p in range(2):
                lb = 2 * i + p
                for t in range(2):
                    wait_gather(t, rbufs[p][t], sems[p][t])
                    acc = compute_chunk(lb, t, rbufs[p][t], acc)

                    @pl.when(lb + 2 < bw)
                    def _():
                        start_gather(lb + 2, t, rbufs[p][t], sems[p][t])
                accbuf[lb] = acc
                acc = jnp.zeros((16,), jnp.float32)
            return acc

        lax.fori_loop(0, bw // 2, body, jnp.zeros((16,), jnp.float32))

        for half in range(bw // 16):
            base = half * 16
            d = jnp.zeros((16,), jnp.float32)
            for l in range(16):
                d = d + plsc.load_gather(accbuf, [base + lanes, col(l)])
            dn_v[pl.ds(base, 16)] = d

        for half in range(bw // 16):
            for r in range(16):
                lb = half * 16 + r
                q = crows_v[lb, pl.ds(0, 16)] * trows_v[lb, pl.ds(0, 16)]
                q = q + crows_v[lb, pl.ds(16, 16)] * trows_v[lb, pl.ds(16, 16)]
                q = q + crows_v[lb, pl.ds(32, 16)] * trows_v[lb, pl.ds(32, 16)]
                q = q + crows_v[lb, pl.ds(48, 16)] * trows_v[lb, pl.ds(48, 16)]
                qbuf[r] = q
            d = jnp.zeros((16,), jnp.float32)
            for l in range(16):
                d = d + plsc.load_gather(qbuf, [lanes, col(l)])
            sc_v[pl.ds(half * 16, 16)] = d

        pltpu.sync_copy(sc_v, scores_hbm.at[pl.ds(base_b, bw)])
        pltpu.sync_copy(dn_v, denom_hbm.at[pl.ds(base_b, bw)])

    return sc_kernel


_sc_kernel = _sc_kernel_make()


def _finish(s_ref, d_ref, o_ref):
    nll = -jnp.mean(s_ref[...] - jnp.log(d_ref[...]))
    o_ref[...] = jnp.full((8, 128), nll, jnp.float32)


_finish_call = pl.pallas_call(
    _finish,
    out_shape=jax.ShapeDtypeStruct((8, 128), jnp.float32),
)


@jax.jit
def kernel(center_words, target_words, all_vocabs, V, U):
    cidx = center_words.reshape(-1).astype(jnp.int32)
    tidx = target_words.reshape(-1).astype(jnp.int32)
    av = all_vocabs.astype(jnp.int32)
    scores, denom = _sc_kernel(cidx, tidx, av, V, U, U.astype(jnp.bfloat16))
    out = _finish_call(scores.reshape(8, 128), denom.reshape(8, 128))
    return out[0, 0]


# speedup vs baseline: 1.0644x; 1.0644x over previous
import functools

import jax
import jax.numpy as jnp
from jax import lax
from jax.experimental import pallas as pl
from jax.experimental.pallas import tpu as pltpu
from jax.experimental.pallas import tpu_sc as plsc

B = 1024
K = 1000
EMB = 64
C0 = 512
C1 = K - C0


def _sc_kernel_make():
    info = plsc.get_sparse_core_info()
    nc, ns = info.num_cores, info.num_subcores
    nw = nc * ns
    bw = B // nw

    mesh = plsc.VectorSubcoreMesh(core_axis_name="c", subcore_axis_name="s")

    @functools.partial(
        pl.kernel,
        mesh=mesh,
        compiler_params=pltpu.CompilerParams(
            needs_layout_passes=False, use_tc_tiling_on_sc=False),
        out_type=[
            jax.ShapeDtypeStruct((B,), jnp.float32),
            jax.ShapeDtypeStruct((B,), jnp.float32),
        ],
        scratch_types=[
            pltpu.VMEM((bw,), jnp.int32),
            pltpu.VMEM((bw,), jnp.int32),
            pltpu.VMEM((bw * K,), jnp.int32),
            pltpu.VMEM((bw, EMB), jnp.bfloat16),
            pltpu.VMEM((bw, EMB), jnp.bfloat16),
            pltpu.VMEM((bw, EMB), jnp.float32),
            pltpu.VMEM((bw, EMB), jnp.float32),
            pltpu.VMEM((C0, EMB), jnp.bfloat16),
            pltpu.VMEM((C0, EMB), jnp.bfloat16),
            pltpu.VMEM((16, 16), jnp.float32),
            pltpu.VMEM((bw, 16), jnp.float32),
            pltpu.VMEM((bw,), jnp.float32),
            pltpu.VMEM((bw,), jnp.float32),
            pltpu.SemaphoreType.DMA,
            pltpu.SemaphoreType.DMA,
            pltpu.SemaphoreType.DMA,
        ],
    )
    def sc_kernel(cidx_hbm, tidx_hbm, av_hbm, v16_hbm, u16_hbm,
                  scores_hbm, denom_hbm,
                  cidx_v, tidx_v, av_v, crows16_v, trows16_v, crows_v, trows_v,
                  rbuf0, rbuf1, qbuf, accbuf, sc_v, dn_v,
                  sem_s, sem0, sem1):
        wid = lax.axis_index("s") * nc + lax.axis_index("c")
        base_b = wid * bw
        lanes = lax.iota(jnp.int32, 16)

        def col(l):
            return jnp.full((16,), l, jnp.int32)

        cp_c = pltpu.make_async_copy(cidx_hbm.at[pl.ds(base_b, bw)], cidx_v, sem_s)
        cp_t = pltpu.make_async_copy(tidx_hbm.at[pl.ds(base_b, bw)], tidx_v, sem_s)
        cp_a = pltpu.make_async_copy(av_hbm.at[pl.ds(base_b * K, bw * K)], av_v, sem_s)
        cp_c.start(); cp_t.start(); cp_a.start()
        cp_c.wait(); cp_t.wait(); cp_a.wait()
        cp_cr = pltpu.make_async_copy(v16_hbm.at[cidx_v], crows16_v, sem_s)
        cp_tr = pltpu.make_async_copy(u16_hbm.at[tidx_v], trows16_v, sem_s)
        cp_cr.start(); cp_tr.start()

        rbufs = (rbuf0, rbuf1)
        sems = (sem0, sem1)

        def start_gather(lb, t, buf, sem):
            if t == 0:
                src = u16_hbm.at[av_v.at[pl.ds(lb * K, C0)]]
                pltpu.make_async_copy(src, buf, sem).start()
            else:
                src = u16_hbm.at[av_v.at[pl.ds(lb * K + C0, C1)]]
                pltpu.make_async_copy(src, buf.at[pl.ds(0, C1)], sem).start()

        def wait_gather(t, buf, sem):
            if t == 0:
                src = u16_hbm.at[av_v.at[pl.ds(0, C0)]]
                pltpu.make_async_copy(src, buf, sem).wait()
            else:
                src = u16_hbm.at[av_v.at[pl.ds(C0, C1)]]
                pltpu.make_async_copy(src, buf.at[pl.ds(0, C1)], sem).wait()

        start_gather(0, 0, rbuf0, sem0)
        start_gather(0, 1, rbuf1, sem1)
        cp_cr.wait(); cp_tr.wait()

        hi_mask = jnp.full((16,), 0xFFFF0000, jnp.uint32)

        def unpack_bf16(v32):
            w = plsc.bitcast(v32, jnp.uint32)
            even = plsc.bitcast(w << 16, jnp.float32)
            odd = plsc.bitcast(w & hi_mask, jnp.float32)
            return even, odd

        for lb in range(bw):
            for half, off in ((0, 0), (1, 32)):
                ev, od = unpack_bf16(crows16_v[lb, pl.ds(off, 32)])
                crows_v[lb, pl.ds(off, 16)] = ev
                crows_v[lb, pl.ds(off + 16, 16)] = od
                ev, od = unpack_bf16(trows16_v[lb, pl.ds(off, 32)])
                trows_v[lb, pl.ds(off, 16)] = ev
                trows_v[lb, pl.ds(off + 16, 16)] = od

        def compute_chunk(lb, t, rbuf, acc):
            ce0 = crows_v[lb, pl.ds(0, 16)]
            co0 = crows_v[lb, pl.ds(16, 16)]
            ce1 = crows_v[lb, pl.ds(32, 16)]
            co1 = crows_v[lb, pl.ds(48, 16)]

            def group(gi, acc):
                for r in range(16):
                    row = gi * 16 + r
                    e0, o0 = unpack_bf16(rbuf[row, pl.ds(0, 32)])
                    e1, o1 = unpack_bf16(rbuf[row, pl.ds(32, 32)])
                    q = e0 * ce0
                    q = q + o0 * co0
                    q = q + e1 * ce1
                    q = q + o1 * co1
                    qbuf[r] = q
                d = jnp.zeros((16,), jnp.float32)
                for l in range(16):
                    d = d + plsc.load_gather(qbuf, [lanes, col(l)])
                e = jnp.exp(d)
                if t == 1:
                    e = jnp.where(gi * 16 + lanes < C1, e, jnp.float32(0.0))
                return acc + e

            ngroups = C0 // 16 if t == 0 else (C1 + 15) // 16
            return lax.fori_loop(0, ngroups, group, acc)

        def body(i, acc):
            lb = i
            for t in range(2):
                wait_gather(t, rbufs[t], sems[t])
                acc = compute_chunk(lb, t, rbufs[t], acc)

                @pl.when(lb + 1 < bw)
                def _():
                    start_gather(lb + 1, t, rbufs[t], sems[t])
            accbuf[lb] = acc
            return jnp.zeros((16,), jnp.float32)

        lax.fori_loop(0, bw, body, jnp.zeros((16,), jnp.float32))

        for half in range(bw // 16):
            base = half * 16
            d = jnp.zeros((16,), jnp.float32)
            for l in range(16):
                d = d + plsc.load_gather(accbuf, [base + lanes, col(l)])
            dn_v[pl.ds(base, 16)] = d

        for half in range(bw // 16):
            for r in range(16):
                lb = half * 16 + r
                q = crows_v[lb, pl.ds(0, 16)] * trows_v[lb, pl.ds(0, 16)]
                q = q + crows_v[lb, pl.ds(16, 16)] * trows_v[lb, pl.ds(16, 16)]
                q = q + crows_v[lb, pl.ds(32, 16)] * trows_v[lb, pl.ds(32, 16)]
                q = q + crows_v[lb, pl.ds(48, 16)] * trows_v[lb, pl.ds(48, 16)]
                qbuf[r] = q
            d = jnp.zeros((16,), jnp.float32)
            for l in range(16):
                d = d + plsc.load_gather(qbuf, [lanes, col(l)])
            sc_v[pl.ds(half * 16, 16)] = d

        pltpu.sync_copy(sc_v, scores_hbm.at[pl.ds(base_b, bw)])
        pltpu.sync_copy(dn_v, denom_hbm.at[pl.ds(base_b, bw)])

    return sc_kernel


_sc_kernel = _sc_kernel_make()


def _finish(s_ref, d_ref, o_ref):
    nll = -jnp.mean(s_ref[...] - jnp.log(d_ref[...]))
    o_ref[...] = jnp.full((8, 128), nll, jnp.float32)


_finish_call = pl.pallas_call(
    _finish,
    out_shape=jax.ShapeDtypeStruct((8, 128), jnp.float32),
)


@jax.jit
def kernel(center_words, target_words, all_vocabs, V, U):
    cidx = center_words.reshape(-1).astype(jnp.int32)
    tidx = target_words.reshape(-1).astype(jnp.int32)
    av = all_vocabs.astype(jnp.int32).reshape(-1)
    scores, denom = _sc_kernel(cidx, tidx, av,
                               V.astype(jnp.bfloat16), U.astype(jnp.bfloat16))
    out = _finish_call(scores.reshape(8, 128), denom.reshape(8, 128))
    return out[0, 0]
